# baseline (speedup 1.0000x reference)
import functools

import jax
import jax.numpy as jnp
from jax import lax
from jax.experimental import pallas as pl
from jax.experimental.pallas import tpu as pltpu
from jax.experimental.pallas import tpu_sc as plsc

NUM_CORES = 2
NUM_SUBCORES = 16
LANES = 16
NW = NUM_CORES * NUM_SUBCORES

BATCH = 16384
DIM = 64
CHUNK = 128
B_PER_W = BATCH // NW
N_CHUNKS = B_PER_W // CHUNK
COL_VREGS = DIM // LANES
ROW_GROUPS = B_PER_W // LANES


def _shuffle(x, idx):
    return lax.gather(
        x, idx[:, None],
        dimension_numbers=lax.GatherDimensionNumbers(
            offset_dims=(), collapsed_slice_dims=(0,), start_index_map=(0,)),
        slice_sizes=(1,),
        mode=lax.GatherScatterMode.PROMISE_IN_BOUNDS)


def _vsqrt(x):
    bits = lax.bitcast_convert_type(x, jnp.int32)
    y = lax.bitcast_convert_type(jnp.int32(0x5F3759DF) - (bits >> 1),
                                 jnp.float32)
    for _ in range(3):
        y = y * (1.5 - 0.5 * x * y * y)
    return jnp.where(x > 0.0, x * y, 0.0)


def _body(hidx_hbm, ridx_hbm, tidx_hbm, etab_hbm, rtab_hbm, out_hbm,
          hidx, ridx, tidx, hrows, rrows, trows, outv, sem):
    wid = lax.axis_index("s") * NUM_CORES + lax.axis_index("c")
    base = wid * B_PER_W

    pltpu.sync_copy(hidx_hbm.at[wid], hidx)
    pltpu.sync_copy(ridx_hbm.at[wid], ridx)
    pltpu.sync_copy(tidx_hbm.at[wid], tidx)

    copies = []
    for j in range(N_CHUNKS):
        dst = pl.ds(j * CHUNK, CHUNK)
        copies.append(pltpu.async_copy(etab_hbm.at[hidx.at[j]],
                                       hrows.at[dst], sem))
        copies.append(pltpu.async_copy(rtab_hbm.at[ridx.at[j]],
                                       rrows.at[dst], sem))
        copies.append(pltpu.async_copy(etab_hbm.at[tidx.at[j]],
                                       trows.at[dst], sem))
    for c in copies:
        c.wait()

    lane = lax.iota(jnp.int32, LANES)

    def group(g, _):
        sums = jnp.zeros((LANES,), jnp.float32)
        for i in range(LANES):
            row = g * LANES + i
            s = None
            for c in range(COL_VREGS):
                cols = pl.ds(c * LANES, LANES)
                h = hrows[row, cols]
                r = rrows[row, cols]
                t = trows[row, cols]
                d = (h + r) - t
                sq = d * d
                s = sq if s is None else s + sq
            for k in (8, 4, 2, 1):
                s = s + _shuffle(s, lane ^ k)
            sums = jnp.where(lane == i, s, sums)
        outv[pl.ds(g * LANES, LANES)] = _vsqrt(sums)
        return ()

    lax.fori_loop(0, ROW_GROUPS, group, (), unroll=False)

    pltpu.sync_copy(outv, out_hbm.at[pl.ds(base, B_PER_W)])


@jax.jit
def _trans_e(hidx, ridx, tidx, etab, rtab):
    mesh = plsc.VectorSubcoreMesh(core_axis_name="c", subcore_axis_name="s")
    f = functools.partial(
        pl.kernel,
        mesh=mesh,
        compiler_params=pltpu.CompilerParams(use_tc_tiling_on_sc=False),
        out_type=jax.ShapeDtypeStruct((BATCH,), jnp.float32),
        scratch_types=[
            pltpu.VMEM((N_CHUNKS, CHUNK), jnp.int32),
            pltpu.VMEM((N_CHUNKS, CHUNK), jnp.int32),
            pltpu.VMEM((N_CHUNKS, CHUNK), jnp.int32),
            pltpu.VMEM((B_PER_W, DIM), jnp.float32),
            pltpu.VMEM((B_PER_W, DIM), jnp.float32),
            pltpu.VMEM((B_PER_W, DIM), jnp.float32),
            pltpu.VMEM((B_PER_W,), jnp.float32),
            pltpu.SemaphoreType.DMA,
        ],
    )(_body)
    return f(hidx, ridx, tidx, etab, rtab)


def kernel(head, relation, tail, entity_table, relation_table):
    hidx = head.reshape(NW, N_CHUNKS, CHUNK)
    ridx = relation.reshape(NW, N_CHUNKS, CHUNK)
    tidx = tail.reshape(NW, N_CHUNKS, CHUNK)
    return _trans_e(hidx, ridx, tidx, entity_table, relation_table)


# speedup vs baseline: 1.6731x; 1.6731x over previous
import functools

import jax
import jax.numpy as jnp
from jax import lax
from jax.experimental import pallas as pl
from jax.experimental.pallas import tpu as pltpu
from jax.experimental.pallas import tpu_sc as plsc

NUM_CORES = 2
NUM_SUBCORES = 16
LANES = 16
NW = NUM_CORES * NUM_SUBCORES

BATCH = 16384
DIM = 64
B_PER_W = BATCH // NW
CHUNK = 128
N_CHUNKS = B_PER_W // CHUNK
COL_VREGS = DIM // LANES
GROUPS_PER_CHUNK = CHUNK // LANES


def _shuffle(x, idx):
    return lax.gather(
        x, idx[:, None],
        dimension_numbers=lax.GatherDimensionNumbers(
            offset_dims=(), collapsed_slice_dims=(0,), start_index_map=(0,)),
        slice_sizes=(1,),
        mode=lax.GatherScatterMode.PROMISE_IN_BOUNDS)


def _vsqrt(x):
    bits = lax.bitcast_convert_type(x, jnp.int32)
    y = lax.bitcast_convert_type(jnp.int32(0x5F3759DF) - (bits >> 1),
                                 jnp.float32)
    for _ in range(3):
        y = y * (1.5 - 0.5 * x * y * y)
    return jnp.where(x > 0.0, x * y, 0.0)


def _body(head_hbm, rel_hbm, tail_hbm, etab_hbm, rtab_hbm, out_hbm,
          hsm, rsm, tsm, hbuf, rbuf, tbuf, outv, sem0, sem1):
    wid = lax.axis_index("s") * NUM_CORES + lax.axis_index("c")
    base = wid * B_PER_W

    pltpu.sync_copy(head_hbm.at[pl.ds(base, B_PER_W)], hsm)
    pltpu.sync_copy(rel_hbm.at[pl.ds(base, B_PER_W)], rsm)
    pltpu.sync_copy(tail_hbm.at[pl.ds(base, B_PER_W)], tsm)

    sems = (sem0, sem1)

    def issue_chunk(c, par):
        def grp(g, _):
            pos = c * CHUNK + g * LANES
            hv = hsm[pl.ds(pos, LANES)]
            rv = rsm[pl.ds(pos, LANES)]
            tv = tsm[pl.ds(pos, LANES)]
            for i in range(LANES):
                j = g * LANES + i
                pltpu.async_copy(etab_hbm.at[hv[i]],
                                 hbuf.at[par, j], sems[par])
                pltpu.async_copy(rtab_hbm.at[rv[i]],
                                 rbuf.at[par, j], sems[par])
                pltpu.async_copy(etab_hbm.at[tv[i]],
                                 tbuf.at[par, j], sems[par])
            return ()
        lax.fori_loop(0, GROUPS_PER_CHUNK, grp, ())

    def drain_chunk(par):
        pltpu.make_async_copy(etab_hbm.at[pl.ds(0, CHUNK)],
                              hbuf.at[par], sems[par]).wait()
        pltpu.make_async_copy(rtab_hbm.at[pl.ds(0, CHUNK)],
                              rbuf.at[par], sems[par]).wait()
        pltpu.make_async_copy(etab_hbm.at[pl.ds(0, CHUNK)],
                              tbuf.at[par], sems[par]).wait()

    lane = lax.iota(jnp.int32, LANES)

    def compute_chunk(c, par):
        def group(g, _):
            sums = jnp.zeros((LANES,), jnp.float32)
            for i in range(LANES):
                row = g * LANES + i
                s = None
                for k in range(COL_VREGS):
                    cols = pl.ds(k * LANES, LANES)
                    h = hbuf[par, row, cols]
                    r = rbuf[par, row, cols]
                    t = tbuf[par, row, cols]
                    d = (h + r) - t
                    sq = d * d
                    s = sq if s is None else s + sq
                for k in (8, 4, 2, 1):
                    s = s + _shuffle(s, lane ^ k)
                sums = jnp.where(lane == i, s, sums)
            outv[pl.ds(c * CHUNK + g * LANES, LANES)] = _vsqrt(sums)
            return ()
        lax.fori_loop(0, GROUPS_PER_CHUNK, group, ())

    issue_chunk(0, 0)
    for c in range(N_CHUNKS):
        if c + 1 < N_CHUNKS:
            issue_chunk(c + 1, (c + 1) % 2)
        drain_chunk(c % 2)
        compute_chunk(c, c % 2)

    pltpu.sync_copy(outv, out_hbm.at[pl.ds(base, B_PER_W)])


@jax.jit
def _trans_e(head, relation, tail, etab, rtab):
    mesh = plsc.VectorSubcoreMesh(core_axis_name="c", subcore_axis_name="s")
    f = functools.partial(
        pl.kernel,
        mesh=mesh,
        out_type=jax.ShapeDtypeStruct((BATCH,), jnp.float32),
        scratch_types=[
            pltpu.VMEM((B_PER_W,), jnp.int32),
            pltpu.VMEM((B_PER_W,), jnp.int32),
            pltpu.VMEM((B_PER_W,), jnp.int32),
            pltpu.VMEM((2, CHUNK, DIM), jnp.float32),
            pltpu.VMEM((2, CHUNK, DIM), jnp.float32),
            pltpu.VMEM((2, CHUNK, DIM), jnp.float32),
            pltpu.VMEM((B_PER_W,), jnp.float32),
            pltpu.SemaphoreType.DMA,
            pltpu.SemaphoreType.DMA,
        ],
    )(_body)
    return f(head, relation, tail, etab, rtab)


def kernel(head, relation, tail, entity_table, relation_table):
    return _trans_e(head, relation, tail, entity_table, relation_table)


# speedup vs baseline: 1.6732x; 1.0001x over previous
import functools

import jax
import jax.numpy as jnp
from jax import lax
from jax.experimental import pallas as pl
from jax.experimental.pallas import tpu as pltpu
from jax.experimental.pallas import tpu_sc as plsc

NUM_CORES = 2
NUM_SUBCORES = 16
LANES = 16
NW = NUM_CORES * NUM_SUBCORES

BATCH = 16384
DIM = 64
B_PER_W = BATCH // NW
CHUNK = 128
N_CHUNKS = B_PER_W // CHUNK
COL_VREGS = DIM // LANES
GROUPS_PER_CHUNK = CHUNK // LANES
NSEM = 8
ROWS_PER_SEM = 3 * CHUNK // NSEM


def _shuffle(x, idx):
    return lax.gather(
        x, idx[:, None],
        dimension_numbers=lax.GatherDimensionNumbers(
            offset_dims=(), collapsed_slice_dims=(0,), start_index_map=(0,)),
        slice_sizes=(1,),
        mode=lax.GatherScatterMode.PROMISE_IN_BOUNDS)


def _vsqrt(x):
    bits = lax.bitcast_convert_type(x, jnp.int32)
    y = lax.bitcast_convert_type(jnp.int32(0x5F3759DF) - (bits >> 1),
                                 jnp.float32)
    for _ in range(3):
        y = y * (1.5 - 0.5 * x * y * y)
    return jnp.where(x > 0.0, x * y, 0.0)


def _body(head_hbm, rel_hbm, tail_hbm, etab_hbm, rtab_hbm, out_hbm,
          hsm, rsm, tsm, hbuf, rbuf, tbuf, outv, *sems):
    wid = lax.axis_index("s") * NUM_CORES + lax.axis_index("c")
    base = wid * B_PER_W

    pltpu.sync_copy(head_hbm.at[pl.ds(base, B_PER_W)], hsm)
    pltpu.sync_copy(rel_hbm.at[pl.ds(base, B_PER_W)], rsm)
    pltpu.sync_copy(tail_hbm.at[pl.ds(base, B_PER_W)], tsm)

    def issue_chunk(c, par):
        def grp(g, _):
            pos = c * CHUNK + g * LANES
            hv = hsm[pl.ds(pos, LANES)]
            rv = rsm[pl.ds(pos, LANES)]
            tv = tsm[pl.ds(pos, LANES)]
            for i in range(LANES):
                j = g * LANES + i
                q = par * NSEM
                pltpu.async_copy(etab_hbm.at[hv[i]],
                                 hbuf.at[par, j], sems[q + (3 * i) % NSEM])
                pltpu.async_copy(rtab_hbm.at[rv[i]],
                                 rbuf.at[par, j], sems[q + (3 * i + 1) % NSEM])
                pltpu.async_copy(etab_hbm.at[tv[i]],
                                 tbuf.at[par, j], sems[q + (3 * i + 2) % NSEM])
            return ()
        lax.fori_loop(0, GROUPS_PER_CHUNK, grp, ())

    def drain_chunk(par):
        for s in range(NSEM):
            pltpu.make_async_copy(etab_hbm.at[pl.ds(0, ROWS_PER_SEM)],
                                  hbuf.at[par, pl.ds(0, ROWS_PER_SEM)],
                                  sems[par * NSEM + s]).wait()

    lane = lax.iota(jnp.int32, LANES)

    def compute_chunk(c, par):
        def group(g, _):
            sums = jnp.zeros((LANES,), jnp.float32)
            for i in range(LANES):
                row = g * LANES + i
                s = None
                for k in range(COL_VREGS):
                    cols = pl.ds(k * LANES, LANES)
                    h = hbuf[par, row, cols]
                    r = rbuf[par, row, cols]
                    t = tbuf[par, row, cols]
                    d = (h + r) - t
                    sq = d * d
                    s = sq if s is None else s + sq
                for k in (8, 4, 2, 1):
                    s = s + _shuffle(s, lane ^ k)
                sums = jnp.where(lane == i, s, sums)
            outv[pl.ds(c * CHUNK + g * LANES, LANES)] = _vsqrt(sums)
            return ()
        lax.fori_loop(0, GROUPS_PER_CHUNK, group, ())

    issue_chunk(0, 0)
    for c in range(N_CHUNKS):
        if c + 1 < N_CHUNKS:
            issue_chunk(c + 1, (c + 1) % 2)
        drain_chunk(c % 2)
        compute_chunk(c, c % 2)

    pltpu.sync_copy(outv, out_hbm.at[pl.ds(base, B_PER_W)])


@jax.jit
def _trans_e(head, relation, tail, etab, rtab):
    mesh = plsc.VectorSubcoreMesh(core_axis_name="c", subcore_axis_name="s")
    f = functools.partial(
        pl.kernel,
        mesh=mesh,
        out_type=jax.ShapeDtypeStruct((BATCH,), jnp.float32),
        scratch_types=[
            pltpu.VMEM((B_PER_W,), jnp.int32),
            pltpu.VMEM((B_PER_W,), jnp.int32),
            pltpu.VMEM((B_PER_W,), jnp.int32),
            pltpu.VMEM((2, CHUNK, DIM), jnp.float32),
            pltpu.VMEM((2, CHUNK, DIM), jnp.float32),
            pltpu.VMEM((2, CHUNK, DIM), jnp.float32),
            pltpu.VMEM((B_PER_W,), jnp.float32),
        ] + [pltpu.SemaphoreType.DMA] * (2 * NSEM),
    )(_body)
    return f(head, relation, tail, etab, rtab)


def kernel(head, relation, tail, entity_table, relation_table):
    return _trans_e(head, relation, tail, entity_table, relation_table)
